# baseline (speedup 1.0000x reference)
import jax
import jax.numpy as jnp
from jax.experimental import pallas as pl
from jax.experimental.pallas import tpu as pltpu

_N = 10000
_BM = 400


def _sage_kernel(adj_ref, x_ref, w_ref, out_ref):
    i = pl.program_id(0)
    a = adj_ref[...]
    s = jnp.dot(a, x_ref[...], preferred_element_type=jnp.float32)
    deg = jnp.sum(a, axis=1, keepdims=True)
    neigh = s / (deg + 1.0)
    xi = x_ref[pl.ds(i * _BM, _BM), :]
    nfeat = x_ref.shape[1]
    dims = (((1,), (1,)), ((), ()))
    h = jax.lax.dot_general(xi, w_ref[:, :nfeat], dims,
                            preferred_element_type=jnp.float32)
    h += jax.lax.dot_general(neigh, w_ref[:, nfeat:], dims,
                             preferred_element_type=jnp.float32)
    out_ref[...] = jnp.maximum(h, 0.0)


@jax.jit
def kernel(x, adj, W):
    nfeat = x.shape[1]
    nembed = W.shape[0]

    return pl.pallas_call(
        _sage_kernel,
        grid=(_N // _BM,),
        in_specs=[
            pl.BlockSpec((_BM, _N), lambda i: (i, 0)),
            pl.BlockSpec((_N, nfeat), lambda i: (0, 0)),
            pl.BlockSpec((nembed, 2 * nfeat), lambda i: (0, 0)),
        ],
        out_specs=pl.BlockSpec((_BM, nembed), lambda i: (i, 0)),
        out_shape=jax.ShapeDtypeStruct((_N, nembed), jnp.float32),
        compiler_params=pltpu.CompilerParams(
            dimension_semantics=("parallel",),
        ),
    )(adj, x, W)


# speedup vs baseline: 1.0037x; 1.0037x over previous
import jax
import jax.numpy as jnp
from jax.experimental import pallas as pl
from jax.experimental.pallas import tpu as pltpu

_N = 10000
_BM = 400


def _sage_kernel(adj_ref, x_ref, w_ref, out_ref):
    i = pl.program_id(0)
    a = adj_ref[...]
    s = jnp.dot(a, x_ref[...], preferred_element_type=jnp.float32)
    deg = jnp.sum(a, axis=1, keepdims=True)
    neigh = s / (deg + 1.0)
    xi = x_ref[pl.ds(i * _BM, _BM), :]
    nfeat = x_ref.shape[1]
    dims = (((1,), (1,)), ((), ()))
    h = jax.lax.dot_general(xi, w_ref[:, :nfeat], dims,
                            preferred_element_type=jnp.float32)
    h += jax.lax.dot_general(neigh, w_ref[:, nfeat:], dims,
                             preferred_element_type=jnp.float32)
    out_ref[...] = jnp.maximum(h, 0.0)


@jax.jit
def kernel(x, adj, W):
    nfeat = x.shape[1]
    nembed = W.shape[0]

    return pl.pallas_call(
        _sage_kernel,
        grid=(_N // _BM,),
        in_specs=[
            pl.BlockSpec((_BM, _N), lambda i: (i, 0)),
            pl.BlockSpec((_N, nfeat), lambda i: (0, 0)),
            pl.BlockSpec((nembed, 2 * nfeat), lambda i: (0, 0)),
        ],
        out_specs=pl.BlockSpec((_BM, nembed), lambda i: (i, 0)),
        out_shape=jax.ShapeDtypeStruct((_N, nembed), jnp.float32),
        compiler_params=pltpu.CompilerParams(
            dimension_semantics=("arbitrary",),
        ),
    )(adj, x, W)
